# Initial kernel scaffold; baseline (speedup 1.0000x reference)
import jax
import jax.numpy as jnp
from jax.experimental import pallas as pl


def kernel(x, edge_index, Wc1, bc1, Wc2, bc2, W1, b1, W2, b2, W3, b3, W4, b4):
    raise NotImplementedError("write your pallas kernel here")


# speedup vs baseline: 6.4590x; 6.4590x over previous
import functools

import jax
import jax.numpy as jnp
from jax.experimental import pallas as pl


def _net_kernel(x_ref, ei_ref, wc1_ref, bc1_ref, wc2_ref, bc2_ref,
                w1_ref, b1_ref, w2_ref, b2_ref, w3_ref, b3_ref,
                w4_ref, b4_ref, out_ref):
    f32 = jnp.float32
    x = x_ref[...]
    ei = ei_ref[...]

    sl = jax.lax.broadcasted_iota(jnp.int32, (1, 4), 1)
    src = jnp.concatenate([ei[0:1, :], sl], axis=1)
    dst = jnp.concatenate([ei[1:2, :], sl], axis=1)

    nodes = jax.lax.broadcasted_iota(jnp.int32, (4, 16), 0)
    S = (jnp.broadcast_to(src, (4, 16)) == nodes).astype(f32)
    D = (jnp.broadcast_to(dst, (4, 16)) == nodes).astype(f32)

    deg = jnp.sum(D, axis=1, keepdims=True)
    dinv = jnp.where(deg > 0, 1.0 / jnp.sqrt(deg), 0.0)
    dinv_src = jnp.sum(dinv * S, axis=0, keepdims=True)
    dinv_dst = jnp.sum(dinv * D, axis=0, keepdims=True)
    norm = dinv_src * dinv_dst

    A = jnp.dot(D * norm, S.T, preferred_element_type=f32, precision=jax.lax.Precision.HIGHEST)

    h1 = jnp.dot(A, jnp.dot(x, wc1_ref[...], preferred_element_type=f32, precision=jax.lax.Precision.HIGHEST),
                 preferred_element_type=f32, precision=jax.lax.Precision.HIGHEST) + bc1_ref[...]
    h2 = jnp.dot(A, jnp.dot(h1, wc2_ref[...], preferred_element_type=f32, precision=jax.lax.Precision.HIGHEST),
                 preferred_element_type=f32, precision=jax.lax.Precision.HIGHEST) + bc2_ref[...]

    x1 = jnp.concatenate([h2[0:1, :], h2[1:2, :], h2[2:3, :], h2[3:4, :]],
                         axis=1)

    x2 = jnp.dot(x1, w1_ref[...], preferred_element_type=f32, precision=jax.lax.Precision.HIGHEST) + b1_ref[...]
    o = jnp.dot(x2, w2_ref[...], preferred_element_type=f32, precision=jax.lax.Precision.HIGHEST) + b2_ref[...]
    o = jnp.dot(o, w3_ref[...], preferred_element_type=f32, precision=jax.lax.Precision.HIGHEST) + b3_ref[...]
    o = o + x1 + x2
    out_ref[...] = jnp.dot(o, w4_ref[...], preferred_element_type=f32, precision=jax.lax.Precision.HIGHEST) \
        + b4_ref[...]


@functools.partial(jax.jit, static_argnames=())
def _run(x, edge_index, Wc1, bc1, Wc2, bc2, W1, b1, W2, b2, W3, b3, W4, b4):
    out = pl.pallas_call(
        _net_kernel,
        out_shape=jax.ShapeDtypeStruct((1, 1), jnp.float32),
    )(x, edge_index,
      Wc1, bc1.reshape(1, -1),
      Wc2, bc2.reshape(1, -1),
      W1, b1.reshape(1, -1),
      W2, b2.reshape(1, -1),
      W3, b3.reshape(1, -1),
      W4, b4.reshape(1, -1))
    return out.reshape(1)


def kernel(x, edge_index, Wc1, bc1, Wc2, bc2, W1, b1, W2, b2, W3, b3, W4, b4):
    return _run(x, edge_index, Wc1, bc1, Wc2, bc2,
                W1, b1, W2, b2, W3, b3, W4, b4)
